# baseline (speedup 1.0000x reference)
import jax
import jax.numpy as jnp
import numpy as np
from jax.experimental import pallas as pl

_LEVEL_HW = [(16, 16), (32, 32), (64, 64), (128, 128)]
_LEVEL_FILTER = [0.25, 0.5, 1.0, 1.0]
_LAYER_FILTER = [1.0, 0.8, 0.6, 0.6, 0.4, 0.2]
_LEVEL_BLK = [256, 1024, 2048, 2048]


def _score_body(f_ref, u_ref, w_ref, o_ref):
    fm = f_ref[0]
    mod = fm + fm * u_ref[0].reshape(-1, 1)
    o_ref[...] = jnp.dot(mod, w_ref[...],
                         preferred_element_type=jnp.float32)[:, 0][None, None]


def _level_score(fml, upa, W_cls, blk):
    B, n, D = fml.shape
    nblk = n // blk
    up3 = upa.reshape(B * nblk, 1, blk)
    out = pl.pallas_call(
        _score_body,
        grid=(B, nblk),
        in_specs=[pl.BlockSpec((1, blk, D), lambda b, i: (b, i, 0)),
                  pl.BlockSpec((1, 1, blk), lambda b, i: (b * nblk + i, 0, 0)),
                  pl.BlockSpec((D, 1), lambda b, i: (0, 0))],
        out_specs=pl.BlockSpec((1, 1, blk), lambda b, i: (b * nblk + i, 0, 0)),
        out_shape=jax.ShapeDtypeStruct((B * nblk, 1, blk), jnp.float32),
    )(fml, up3, W_cls)
    return out.reshape(B, n)


def kernel(features, W_cls, b_cls, alpha):
    B, N, D = features.shape
    splits = [int(s) for s in np.cumsum([h * w for h, w in _LEVEL_HW])[:-1]]
    feat_levels = jnp.split(features, splits, axis=1)

    prev_score = None
    sel_scores, sel_xy, sel_lvl = [], [], []
    for li, (h, w) in enumerate(_LEVEL_HW):
        n = h * w
        if li == 0:
            upa = jnp.zeros((B, n), dtype=jnp.float32)
        else:
            ph, pw = _LEVEL_HW[li - 1]
            up = prev_score.reshape(B, ph, pw)
            up = jnp.repeat(jnp.repeat(up, 2, axis=1), 2, axis=2) * alpha[li - 1]
            upa = up.reshape(B, n)
        score = _level_score(feat_levels[li], upa, W_cls, _LEVEL_BLK[li]) + b_cls[0]
        prev_score = score
        k = int(n * _LEVEL_FILTER[li])
        topv, topi = jax.lax.top_k(score, k)
        ii = topi // w
        jj = topi % w
        x = (jj.astype(jnp.float32) + 0.5) / w
        y = (ii.astype(jnp.float32) + 0.5) / h
        sel_scores.append(topv)
        sel_xy.append(jnp.stack([x, y], axis=-1))
        sel_lvl.append(jnp.full((B, k), li, dtype=jnp.int32))
    all_scores = jnp.concatenate(sel_scores, axis=1)
    all_xy = jnp.concatenate(sel_xy, axis=1)
    all_lvl = jnp.concatenate(sel_lvl, axis=1)
    order = jnp.argsort(-all_scores, axis=1)
    sorted_scores = jnp.take_along_axis(all_scores, order, axis=1)
    K = all_scores.shape[1]
    per_layer_idx = tuple(order[:, : int(K * r)] for r in _LAYER_FILTER)
    return (sorted_scores, all_xy, all_lvl) + per_layer_idx


# speedup vs baseline: 1.0662x; 1.0662x over previous
import functools

import jax
import jax.numpy as jnp
import numpy as np
from jax import lax
from jax.experimental import pallas as pl
from jax.experimental.pallas import tpu as pltpu, tpu_sc as plsc

_LEVEL_HW = [(16, 16), (32, 32), (64, 64), (128, 128)]
_LEVEL_FILTER = [0.25, 0.5, 1.0, 1.0]
_LAYER_FILTER = [1.0, 0.8, 0.6, 0.6, 0.4, 0.2]
_LEVEL_BLK = [256, 1024, 2048, 2048]

_N = 21760
_CH = _N // 16
_K_OUT = 21056
_KCH = _K_OUT // 16
_I32MIN = jnp.int32(-2147483648)


def _score_body(f_ref, u_ref, w_ref, o_ref):
    fm = f_ref[0]
    mod = fm + fm * u_ref[0].reshape(-1, 1)
    o_ref[...] = jnp.dot(mod, w_ref[...],
                         preferred_element_type=jnp.float32)[:, 0][None, None]


def _level_score(fml, upa, W_cls, blk):
    B, n, D = fml.shape
    nblk = n // blk
    up3 = upa.reshape(B * nblk, 1, blk)
    out = pl.pallas_call(
        _score_body,
        grid=(B, nblk),
        in_specs=[pl.BlockSpec((1, blk, D), lambda b, i: (b, i, 0)),
                  pl.BlockSpec((1, 1, blk), lambda b, i: (b * nblk + i, 0, 0)),
                  pl.BlockSpec((D, 1), lambda b, i: (0, 0))],
        out_specs=pl.BlockSpec((1, 1, blk), lambda b, i: (b * nblk + i, 0, 0)),
        out_shape=jax.ShapeDtypeStruct((B * nblk, 1, blk), jnp.float32),
    )(fml, up3, W_cls)
    return out.reshape(B, n)


def _digit(vals_f32, shift):
    u = plsc.bitcast(vals_f32, jnp.int32)
    m = lax.shift_right_arithmetic(u, 31)
    key = ~(u ^ (m | _I32MIN))
    return lax.shift_right_logical(key, shift) & 255


def _sc_body(sc_hbm, ss_hbm, ord_hbm, xx_hbm, yy_hbm, Ak, Ap, Bk, Bp, hist, xyb):
    wid = lax.axis_index("s") * 2 + lax.axis_index("c")

    @pl.when(wid < 4)
    def _():
        b = wid
        lane = lax.iota(jnp.int32, 16)
        lane_str = lane * _CH
        lane_h = lane * 256

        pltpu.sync_copy(sc_hbm.at[pl.ds(b * _N, _N)], Ak)

        def initb(c, _):
            Ap[pl.ds(c * 16, 16)] = c * 16 + lane
            return 0
        lax.fori_loop(0, _CH, initb, 0)

        def radix_pass(shift, Ki, Pi, Ko, Po):
            def z(c, _):
                hist[pl.ds(c * 16, 16)] = jnp.zeros((16,), jnp.int32)
                return 0
            lax.fori_loop(0, 256, z, 0)

            ones = jnp.ones((16,), jnp.int32)

            def pa(c, _):
                k = plsc.load_gather(Ki, [lane_str + c])
                d = _digit(k, shift)
                plsc.addupdate_scatter(hist, [lane_h + d], ones)
                return 0
            lax.fori_loop(0, _CH, pa, 0)

            def sc16(dc, carry):
                vs = [hist[pl.ds(l * 256 + dc * 16, 16)] for l in range(16)]
                a = jnp.zeros((16,), jnp.int32)
                accs = []
                for l in range(16):
                    accs.append(a)
                    a = a + vs[l]
                total = a
                g = carry + plsc.cumsum(total) - total
                for l in range(16):
                    hist[pl.ds(l * 256 + dc * 16, 16)] = accs[l] + g
                return carry + jnp.sum(total, axis=0)
            lax.fori_loop(0, 16, sc16, jnp.int32(0))

            def pb(c, _):
                idx = lane_str + c
                k = plsc.load_gather(Ki, [idx])
                p = plsc.load_gather(Pi, [idx])
                d = _digit(k, shift)
                h = lane_h + d
                off = plsc.load_gather(hist, [h])
                plsc.store_scatter(Ko, [off], k)
                plsc.store_scatter(Po, [off], p)
                plsc.store_scatter(hist, [h], off + 1)
                return 0
            lax.fori_loop(0, _CH, pb, 0)

        radix_pass(0, Ak, Ap, Bk, Bp)
        radix_pass(8, Bk, Bp, Ak, Ap)
        radix_pass(16, Ak, Ap, Bk, Bp)
        radix_pass(24, Bk, Bp, Ak, Ap)

        zero = jnp.int32(0)

        def post(c, carry):
            gsel, l0, l1, l2, l3 = carry
            s = Ak[pl.ds(c * 16, 16)]
            t = Ap[pl.ds(c * 16, 16)]
            ge1 = t >= 256
            ge2 = t >= 1280
            ge3 = t >= 5376
            sh = (ge1.astype(jnp.int32) + ge2.astype(jnp.int32)
                  + ge3.astype(jnp.int32)) << 3
            enc = lax.shift_left(jnp.ones((16,), jnp.int32), sh)
            scs = plsc.cumsum(enc)
            cnt = lax.shift_right_logical(scs - enc, sh) & 255
            lb = jnp.where(ge2, jnp.where(ge3, l3, l2), jnp.where(ge1, l1, l0))
            rank = lb + cnt
            kv = jnp.where(ge2, jnp.where(ge3, 16384, 4096),
                           jnp.where(ge1, 512, 64))
            offv = jnp.where(ge2, jnp.where(ge3, 4672, 576),
                             jnp.where(ge1, 64, 0))
            sel = rank < kv
            seli = sel.astype(jnp.int32)
            sx = plsc.cumsum(seli)
            gr = gsel + sx - seli
            gidx = jnp.where(sel, gr, zero)
            concat = offv + rank
            cidx = jnp.where(sel, concat, zero)
            plsc.store_scatter(Ak, [gidx], s, mask=sel)
            plsc.store_scatter(Bp, [gidx], concat, mask=sel)
            plsc.store_scatter(Bk, [cidx], plsc.bitcast(t, jnp.float32),
                               mask=sel)
            tot = jnp.sum(enc, axis=0)
            nsel = jnp.sum(seli, axis=0)
            return (gsel + nsel,
                    l0 + (tot & 255),
                    l1 + (lax.shift_right_logical(tot, 8) & 255),
                    l2 + (lax.shift_right_logical(tot, 16) & 255),
                    l3 + (lax.shift_right_logical(tot, 24) & 255))
        lax.fori_loop(0, _CH, post, (zero, zero, zero, zero, zero))

        pltpu.sync_copy(Ak.at[pl.ds(0, _K_OUT)],
                        ss_hbm.at[pl.ds(b * _K_OUT, _K_OUT)])
        pltpu.sync_copy(Bp.at[pl.ds(0, _K_OUT)],
                        ord_hbm.at[pl.ds(b * _K_OUT, _K_OUT)])

        half = jnp.float32(0.5)

        def xy_sweep(plane, dst_hbm):
            def bodyx(c, _):
                tb = plsc.bitcast(Bk[pl.ds(c * 16, 16)], jnp.int32)
                q = c * 16 + lane
                qge1 = q >= 64
                qge2 = q >= 576
                qge3 = q >= 4672
                lvlq = (qge1.astype(jnp.int32) + qge2.astype(jnp.int32)
                        + qge3.astype(jnp.int32))
                startv = jnp.where(qge2, jnp.where(qge3, 5376, 1280),
                                   jnp.where(qge1, 256, 0))
                logw = 4 + lvlq
                u = tb - startv
                if plane == 0:
                    comp = u & (lax.shift_left(jnp.ones((16,), jnp.int32),
                                               logw) - 1)
                else:
                    comp = lax.shift_right_logical(u, logw)
                invw = plsc.bitcast(lax.shift_left(127 - logw, 23),
                                    jnp.float32)
                xyb[pl.ds(c * 16, 16)] = (comp.astype(jnp.float32) + half) * invw
                return 0
            lax.fori_loop(0, _KCH, bodyx, 0)
            pltpu.sync_copy(xyb, dst_hbm.at[pl.ds(b * _K_OUT, _K_OUT)])

        xy_sweep(0, xx_hbm)
        xy_sweep(1, yy_hbm)


def _sc_select_sort(scores):
    B = scores.shape[0]
    mesh = plsc.VectorSubcoreMesh(core_axis_name="c", subcore_axis_name="s")
    f = pl.kernel(
        _sc_body,
        out_type=[jax.ShapeDtypeStruct((B * _K_OUT,), jnp.float32),
                  jax.ShapeDtypeStruct((B * _K_OUT,), jnp.int32),
                  jax.ShapeDtypeStruct((B * _K_OUT,), jnp.float32),
                  jax.ShapeDtypeStruct((B * _K_OUT,), jnp.float32)],
        mesh=mesh,
        compiler_params=pltpu.CompilerParams(needs_layout_passes=False),
        scratch_types=[pltpu.VMEM((_N,), jnp.float32),
                       pltpu.VMEM((_N,), jnp.int32),
                       pltpu.VMEM((_N,), jnp.float32),
                       pltpu.VMEM((_N,), jnp.int32),
                       pltpu.VMEM((4096,), jnp.int32),
                       pltpu.VMEM((_K_OUT,), jnp.float32)],
    )
    ss, ordr, xx, yy = f(scores.reshape(B * _N))
    return (ss.reshape(B, _K_OUT), ordr.reshape(B, _K_OUT),
            xx.reshape(B, _K_OUT), yy.reshape(B, _K_OUT))


def kernel(features, W_cls, b_cls, alpha):
    B, N, D = features.shape
    splits = [int(s) for s in np.cumsum([h * w for h, w in _LEVEL_HW])[:-1]]
    feat_levels = jnp.split(features, splits, axis=1)

    prev_score = None
    level_scores = []
    for li, (h, w) in enumerate(_LEVEL_HW):
        n = h * w
        if li == 0:
            upa = jnp.zeros((B, n), dtype=jnp.float32)
        else:
            ph, pw = _LEVEL_HW[li - 1]
            up = prev_score.reshape(B, ph, pw)
            up = jnp.repeat(jnp.repeat(up, 2, axis=1), 2, axis=2) * alpha[li - 1]
            upa = up.reshape(B, n)
        score = _level_score(feat_levels[li], upa, W_cls, _LEVEL_BLK[li]) + b_cls[0]
        prev_score = score
        level_scores.append(score)

    scores = jnp.concatenate(level_scores, axis=1)
    sorted_scores, order, xx, yy = _sc_select_sort(scores)
    all_xy = jnp.stack([xx, yy], axis=-1)

    ks = [int(h * w * r) for (h, w), r in zip(_LEVEL_HW, _LEVEL_FILTER)]
    all_lvl = jnp.concatenate(
        [jnp.full((B, k), li, dtype=jnp.int32) for li, k in enumerate(ks)],
        axis=1)
    K = sum(ks)
    per_layer_idx = tuple(order[:, : int(K * r)] for r in _LAYER_FILTER)
    return (sorted_scores, all_xy, all_lvl) + per_layer_idx
